# baseline (speedup 1.0000x reference)
import jax
import jax.numpy as jnp
from jax.experimental import pallas as pl
from jax.experimental.pallas import tpu as pltpu

_MARGIN = 1.0
_NBLK = 5


def _loss_kernel(woT_ref, lab_ref, tab_ref, out_ref, min_acc, lab_acc):
    i = pl.program_id(0)
    B = woT_ref.shape[1]
    Cb = tab_ref.shape[0]
    woT = woT_ref[:]
    x2 = jnp.sum(woT * woT, axis=0, keepdims=True)
    inv = jax.lax.rsqrt(jnp.maximum(x2, 1e-24))
    wnT = woT * (-2.0 * inv)
    xn2 = x2 * (inv * inv)
    rhs = jnp.concatenate([wnT, jnp.ones((1, B), jnp.float32)], axis=0)

    tab = tab_ref[:]
    t2 = jnp.sum(tab * tab, axis=1, keepdims=True)
    lhs = jnp.concatenate([tab, t2], axis=1)
    d2p = jnp.dot(lhs, rhs, preferred_element_type=jnp.float32)

    lab = lab_ref[:]
    rows = jax.lax.broadcasted_iota(jnp.int32, (Cb, B), 0) + i * Cb
    is_lab = rows == lab
    bsum = jnp.sum(jnp.where(is_lab, d2p, 0.0), axis=0, keepdims=True)
    bmin = jnp.min(jnp.where(is_lab, jnp.inf, d2p), axis=0, keepdims=True)

    @pl.when(i == 0)
    def _():
        min_acc[:, :] = bmin
        lab_acc[:, :] = bsum

    @pl.when(i > 0)
    def _():
        min_acc[:, :] = jnp.minimum(min_acc[:, :], bmin)
        lab_acc[:, :] = lab_acc[:, :] + bsum

    @pl.when(i == pl.num_programs(0) - 1)
    def _():
        lab_d2 = lab_acc[:, :] + xn2
        min_d2 = min_acc[:, :] + xn2
        lab_d = lab_d2 * jax.lax.rsqrt(jnp.maximum(lab_d2, 1e-30))
        min_d = min_d2 * jax.lax.rsqrt(jnp.maximum(min_d2, 1e-30))
        s = jnp.sum(lab_d - min_d, axis=1, keepdims=True)
        out_ref[:, :] = _MARGIN + s / B


def kernel(WO, label, table):
    B, D = WO.shape
    C = table.shape[0]
    Cb = C // _NBLK
    out = pl.pallas_call(
        _loss_kernel,
        grid=(_NBLK,),
        in_specs=[
            pl.BlockSpec((D, B), lambda i: (0, 0)),
            pl.BlockSpec((1, B), lambda i: (0, 0)),
            pl.BlockSpec((Cb, D), lambda i: (i, 0)),
        ],
        out_specs=pl.BlockSpec((1, 1), lambda i: (0, 0)),
        out_shape=jax.ShapeDtypeStruct((1, 1), jnp.float32),
        scratch_shapes=[
            pltpu.VMEM((1, B), jnp.float32),
            pltpu.VMEM((1, B), jnp.float32),
        ],
    )(WO.T, label.astype(jnp.int32).reshape(1, B), table)
    return out[0, 0]
